# Initial kernel scaffold; baseline (speedup 1.0000x reference)
import jax
import jax.numpy as jnp
from jax.experimental import pallas as pl


def kernel(x, edge_index, W1, b1, Wih, Whh, bih, bhh, W2, b2, W3, b3, W4, b4, W5, b5, Wfc, bfc):
    raise NotImplementedError("write your pallas kernel here")


# speedup vs baseline: 11.5165x; 11.5165x over previous
import functools

import jax
import jax.numpy as jnp
from jax import lax
from jax.experimental import pallas as pl
from jax.experimental.pallas import tpu as pltpu
from jax.experimental.pallas import tpu_sc as plsc

NC = 2
NS = 16
LANES = 16
CH = 128
BN = 1024


def _sc_mesh():
    return plsc.VectorSubcoreMesh(
        core_axis_name="c", subcore_axis_name="s", num_cores=NC, num_subcores=NS
    )


def _deg_body(nrows, ept, dst_hbm, out_hbm, dvm, hist, pvm, parts_sh):
    c = lax.axis_index("c")
    s = lax.axis_index("s")
    rpt = nrows // NS
    ones = jnp.ones((LANES,), jnp.float32)
    zeros = jnp.zeros((LANES,), jnp.float32)

    @pl.when(c == 0)
    def _():
        pltpu.sync_copy(dst_hbm.at[pl.ds(s * ept, ept)], dvm)

        def zbody(i, _):
            hist[pl.ds(i * LANES, LANES)] = zeros
            return 0

        lax.fori_loop(0, nrows // LANES, zbody, 0)

        def hbody(i, _):
            idx = dvm[pl.ds(i * LANES, LANES)]
            plsc.addupdate_scatter(hist, [idx], ones)
            return 0

        lax.fori_loop(0, ept // LANES, hbody, 0)
        pltpu.sync_copy(hist, parts_sh.at[s])

    plsc.subcore_barrier()

    @pl.when(c == 0)
    def _():
        col0 = s * rpt
        pltpu.sync_copy(parts_sh.at[:, pl.ds(col0, rpt)], pvm)

        def rbody(i, _):
            acc = pvm[0, pl.ds(i * LANES, LANES)]
            for r in range(1, NS):
                acc = acc + pvm[r, pl.ds(i * LANES, LANES)]
            d = jnp.maximum(acc, 0.25)
            ib = plsc.bitcast(d, jnp.int32)
            y = plsc.bitcast(
                jnp.int32(0x5F3759DF) - lax.shift_right_logical(ib, 1), jnp.float32
            )
            for _ in range(4):
                y = y * (1.5 - 0.5 * d * y * y)
            hist[pl.ds(i * LANES, LANES)] = y
            return 0

        lax.fori_loop(0, rpt // LANES, rbody, 0)
        pltpu.sync_copy(hist.at[pl.ds(0, rpt)], out_hbm.at[pl.ds(col0, rpt)])


def _deg_dinv(dst_flat, nrows, ept):
    kfn = pl.kernel(
        functools.partial(_deg_body, nrows, ept),
        out_type=jax.ShapeDtypeStruct((nrows,), jnp.float32),
        mesh=_sc_mesh(),
        scratch_types=[
            pltpu.VMEM((ept,), jnp.int32),
            pltpu.VMEM((nrows,), jnp.float32),
            pltpu.VMEM((NS, nrows // NS), jnp.float32),
            pltpu.VMEM_SHARED((NS, nrows), jnp.float32),
        ],
        compiler_params=pltpu.CompilerParams(needs_layout_passes=False),
    )
    return kfn(dst_flat)


def _agg_body(nrows, nchunk, hc, col_split, xw_hbm, src_hbm, dst_hbm, out_hbm,
              sidx0, sidx1, didx0, didx1, buf0, buf1, accum,
              sem_s0, sem_s1, sem_d0, sem_d1, sem_g0, sem_g1):
    c = lax.axis_index("c")
    s = lax.axis_index("s")
    rpt = nrows // NS
    zeros = jnp.zeros((LANES,), jnp.float32)

    ept = nchunk * CH
    if col_split:
        base = s * ept
        gsrc = xw_hbm.at[c]
    else:
        base = (c * NS + s) * ept
        gsrc = xw_hbm

    def zbody(i, _):
        for k in range(hc // LANES):
            buf0[i, pl.ds(k * LANES, LANES)] = zeros
        return 0

    lax.fori_loop(0, CH, zbody, 0)
    for r in range(rpt // CH):
        pltpu.sync_copy(buf0, accum.at[pl.ds(s * rpt + r * CH, CH)])

    plsc.subcore_barrier()

    def do_pair(j0):
        o0 = base + j0 * CH
        a0 = pltpu.async_copy(src_hbm.at[pl.ds(o0, CH)], sidx0, sem_s0)
        a1 = pltpu.async_copy(src_hbm.at[pl.ds(o0 + CH, CH)], sidx1, sem_s1)
        a2 = pltpu.async_copy(dst_hbm.at[pl.ds(o0, CH)], didx0, sem_d0)
        a3 = pltpu.async_copy(dst_hbm.at[pl.ds(o0 + CH, CH)], didx1, sem_d1)
        a0.wait()
        g0 = pltpu.async_copy(gsrc.at[sidx0], buf0, sem_g0)
        a1.wait()
        g1 = pltpu.async_copy(gsrc.at[sidx1], buf1, sem_g1)
        g0.wait()
        a2.wait()
        pltpu.sync_copy(buf0, accum.at[didx0], add=True)
        g1.wait()
        a3.wait()
        pltpu.sync_copy(buf1, accum.at[didx1], add=True)

    def chunk2(jj, _):
        do_pair(jj * 2)
        return 0

    lax.fori_loop(0, nchunk // 2, chunk2, 0)
    if nchunk % 2:
        o0 = base + (nchunk - 1) * CH
        a0 = pltpu.async_copy(src_hbm.at[pl.ds(o0, CH)], sidx0, sem_s0)
        a2 = pltpu.async_copy(dst_hbm.at[pl.ds(o0, CH)], didx0, sem_d0)
        a0.wait()
        pltpu.async_copy(gsrc.at[sidx0], buf0, sem_g0).wait()
        a2.wait()
        pltpu.sync_copy(buf0, accum.at[didx0], add=True)

    plsc.subcore_barrier()
    pltpu.sync_copy(
        accum.at[pl.ds(s * rpt, rpt)], out_hbm.at[c].at[pl.ds(s * rpt, rpt)]
    )


def _sc_agg(xw, srcp, dstp, nrows, hc, col_split=True):
    epad = srcp.shape[0]
    nchunk = epad // (NS * CH) if col_split else epad // (NC * NS * CH)
    kfn = pl.kernel(
        functools.partial(_agg_body, nrows, nchunk, hc, col_split),
        out_type=jax.ShapeDtypeStruct((NC, nrows, hc), jnp.float32),
        mesh=_sc_mesh(),
        scratch_types=[
            pltpu.VMEM((CH,), jnp.int32),
            pltpu.VMEM((CH,), jnp.int32),
            pltpu.VMEM((CH,), jnp.int32),
            pltpu.VMEM((CH,), jnp.int32),
            pltpu.VMEM((CH, hc), jnp.float32),
            pltpu.VMEM((CH, hc), jnp.float32),
            pltpu.VMEM_SHARED((nrows, hc), jnp.float32),
            pltpu.SemaphoreType.DMA,
            pltpu.SemaphoreType.DMA,
            pltpu.SemaphoreType.DMA,
            pltpu.SemaphoreType.DMA,
            pltpu.SemaphoreType.DMA,
            pltpu.SemaphoreType.DMA,
        ],
        compiler_params=pltpu.CompilerParams(needs_layout_passes=False),
    )
    return kfn(xw, srcp, dstp)


def _mm_split_body(x_ref, dinv_ref, W_ref, o_ref):
    y = jnp.dot(x_ref[...], W_ref[...], preferred_element_type=jnp.float32)
    y = y * dinv_ref[...]
    hc = o_ref.shape[2]
    o_ref[0] = y[:, :hc]
    o_ref[1] = y[:, hc:]


def _mid_body(agg_ref, dinv_ref, b_ref, W_ref, o_ref):
    dinv = dinv_ref[...]
    hin_c = agg_ref.shape[2]
    W = W_ref[...]
    a0 = jnp.maximum(agg_ref[0] * dinv + b_ref[:, :hin_c], 0.0)
    a1 = jnp.maximum(agg_ref[1] * dinv + b_ref[:, hin_c:], 0.0)
    y = (
        jnp.dot(a0, W[:hin_c], preferred_element_type=jnp.float32)
        + jnp.dot(a1, W[hin_c:], preferred_element_type=jnp.float32)
    ) * dinv
    hc = o_ref.shape[2]
    o_ref[0] = y[:, :hc]
    o_ref[1] = y[:, hc:]


def _proj_body(agg_ref, dinv_ref, b_ref, WT_ref, bb_ref, o_ref):
    dinv = dinv_ref[...]
    hin_c = agg_ref.shape[2]
    WT = WT_ref[...]
    a0 = jnp.maximum(agg_ref[0] * dinv + b_ref[:, :hin_c], 0.0)
    a1 = jnp.maximum(agg_ref[1] * dinv + b_ref[:, hin_c:], 0.0)
    o_ref[...] = (
        jnp.dot(a0, WT[:hin_c], preferred_element_type=jnp.float32)
        + jnp.dot(a1, WT[hin_c:], preferred_element_type=jnp.float32)
        + bb_ref[...]
    )


def _mid_full_body(agg_ref, dinv_ref, b_ref, W_ref, o_ref):
    dinv = dinv_ref[...]
    hin_c = agg_ref.shape[2]
    W = W_ref[...]
    a0 = jnp.maximum(agg_ref[0] * dinv + b_ref[:, :hin_c], 0.0)
    a1 = jnp.maximum(agg_ref[1] * dinv + b_ref[:, hin_c:], 0.0)
    o_ref[...] = (
        jnp.dot(a0, W[:hin_c], preferred_element_type=jnp.float32)
        + jnp.dot(a1, W[hin_c:], preferred_element_type=jnp.float32)
    ) * dinv


def _mid_psum_body(agg_ref, dinv_ref, b_ref, W_ref, o_ref):
    dinv = dinv_ref[...]
    a = jnp.maximum((agg_ref[0] + agg_ref[1]) * dinv + b_ref[...], 0.0)
    o_ref[...] = jnp.dot(a, W_ref[...], preferred_element_type=jnp.float32) * dinv


def _fc_body(agg_ref, dinv_ref, b_ref, W_ref, bfc_ref, o_ref):
    dinv = dinv_ref[...]
    a = jnp.maximum((agg_ref[0] + agg_ref[1]) * dinv + b_ref[...], 0.0)
    y = jnp.dot(a, W_ref[...], preferred_element_type=jnp.float32) + bfc_ref[...]
    o_ref[...] = jax.nn.sigmoid(y)


def _row_spec(h):
    return pl.BlockSpec((BN, h), lambda i: (i, 0))


def _half_spec(hc):
    return pl.BlockSpec((2, BN, hc), lambda i: (0, i, 0))


def _full_spec(shape):
    nd = len(shape)
    return pl.BlockSpec(shape, lambda i: (0,) * nd)


def _lstm_body(steps, G_ref, WT_ref, hs_ref, h_sc, c_sc):
    H = hs_ref.shape[1]

    @pl.when(pl.program_id(0) == 0)
    def _():
        h_sc[...] = jnp.zeros_like(h_sc)
        c_sc[...] = jnp.zeros_like(c_sc)

    WT = WT_ref[...]

    def step(t, hc):
        h, c = hc
        g = G_ref[pl.ds(t, 1), :]
        gates = jnp.dot(h, WT, preferred_element_type=jnp.float32) + g
        i = jax.nn.sigmoid(gates[:, 0:H])
        f = jax.nn.sigmoid(gates[:, H:2 * H])
        gg = jnp.tanh(gates[:, 2 * H:3 * H])
        o = jax.nn.sigmoid(gates[:, 3 * H:4 * H])
        c2 = f * c + i * gg
        h2 = o * jnp.tanh(c2)
        hs_ref[pl.ds(t, 1), :] = h2[0:1]
        return (h2, c2)

    hN, cN = lax.fori_loop(0, steps, step, (h_sc[...], c_sc[...]))
    h_sc[...] = hN
    c_sc[...] = cN


def kernel(x, edge_index, W1, b1, Wih, Whh, bih, bhh,
           W2, b2, W3, b3, W4, b4, W5, b5, Wfc, bfc):
    N, D = x.shape
    E = edge_index.shape[1]
    H1 = W1.shape[1]
    H = Whh.shape[1]

    epad = ((E + N + NC * NS * CH - 1) // (NC * NS * CH)) * (NC * NS * CH)
    ept = epad // NS
    nchunk = ept // CH
    nrows = ((N + 1 + NS * CH - 1) // (NS * CH)) * (NS * CH)
    nb = nrows // BN

    loops = jnp.arange(N, dtype=jnp.int32)
    pad = epad - E - N
    srcp = jnp.concatenate([edge_index[0], loops, jnp.zeros((pad,), jnp.int32)])
    dstp = jnp.concatenate([edge_index[1], loops, jnp.full((pad,), N, jnp.int32)])

    xp = jnp.zeros((nrows, D), jnp.float32).at[:N].set(x)
    WihT = Wih.T
    WhhT = Whh.T
    bb = (bih + bhh).reshape(1, 4 * H)
    b1r = b1.reshape(1, -1)
    b2r = b2.reshape(1, -1)
    b3r = b3.reshape(1, -1)
    b4r = b4.reshape(1, -1)
    b5r = b5.reshape(1, -1)
    bfcr = bfc.reshape(1, -1)

    dinv1d = _deg_dinv(dstp, nrows, ept)
    dinv = dinv1d.reshape(nrows, 1)

    dinv_spec = pl.BlockSpec((BN, 1), lambda i: (i, 0))

    def mm_split(a, W):
        hc = W.shape[1] // 2
        return pl.pallas_call(
            _mm_split_body,
            grid=(nb,),
            in_specs=[_row_spec(a.shape[1]), dinv_spec, _full_spec(W.shape)],
            out_specs=_half_spec(hc),
            out_shape=jax.ShapeDtypeStruct((2, nrows, hc), jnp.float32),
        )(a, dinv, W)

    def mm_mid(agg, b, W):
        hin_c = agg.shape[2]
        hc = W.shape[1] // 2
        return pl.pallas_call(
            _mid_body,
            grid=(nb,),
            in_specs=[_half_spec(hin_c), dinv_spec,
                      _full_spec(b.shape), _full_spec(W.shape)],
            out_specs=_half_spec(hc),
            out_shape=jax.ShapeDtypeStruct((2, nrows, hc), jnp.float32),
        )(agg, dinv, b, W)

    xw1 = mm_split(xp, W1)
    agg1 = _sc_agg(xw1, srcp, dstp, nrows, H1 // 2)

    G = pl.pallas_call(
        _proj_body,
        grid=(nb,),
        in_specs=[_half_spec(H1 // 2), dinv_spec, _full_spec(b1r.shape),
                  _full_spec(WihT.shape), _full_spec(bb.shape)],
        out_specs=_row_spec(4 * H),
        out_shape=jax.ShapeDtypeStruct((nrows, 4 * H), jnp.float32),
    )(agg1, dinv, b1r, WihT, bb)

    T = 1000
    hs = pl.pallas_call(
        functools.partial(_lstm_body, T),
        grid=(N // T,),
        in_specs=[pl.BlockSpec((T, 4 * H), lambda i: (i, 0)),
                  _full_spec(WhhT.shape)],
        out_specs=pl.BlockSpec((T, H), lambda i: (i, 0)),
        out_shape=jax.ShapeDtypeStruct((nrows, H), jnp.float32),
        scratch_shapes=[pltpu.VMEM((8, H), jnp.float32),
                        pltpu.VMEM((8, H), jnp.float32)],
    )(G, WhhT)

    xw2 = mm_split(hs, W2)
    agg2 = _sc_agg(xw2, srcp, dstp, nrows, W2.shape[1] // 2)

    xw3 = mm_mid(agg2, b2r, W3)
    agg3 = _sc_agg(xw3, srcp, dstp, nrows, W3.shape[1] // 2)

    xw4 = pl.pallas_call(
        _mid_full_body,
        grid=(nb,),
        in_specs=[_half_spec(W3.shape[1] // 2), dinv_spec,
                  _full_spec(b3r.shape), _full_spec(W4.shape)],
        out_specs=_row_spec(W4.shape[1]),
        out_shape=jax.ShapeDtypeStruct((nrows, W4.shape[1]), jnp.float32),
    )(agg3, dinv, b3r, W4)
    agg4 = _sc_agg(xw4, srcp, dstp, nrows, W4.shape[1], col_split=False)

    xw5 = pl.pallas_call(
        _mid_psum_body,
        grid=(nb,),
        in_specs=[_half_spec(W4.shape[1]), dinv_spec,
                  _full_spec(b4r.shape), _full_spec(W5.shape)],
        out_specs=_row_spec(W5.shape[1]),
        out_shape=jax.ShapeDtypeStruct((nrows, W5.shape[1]), jnp.float32),
    )(agg4, dinv, b4r, W5)
    agg5 = _sc_agg(xw5, srcp, dstp, nrows, W5.shape[1], col_split=False)

    out = pl.pallas_call(
        _fc_body,
        grid=(nb,),
        in_specs=[_half_spec(W5.shape[1]), dinv_spec,
                  _full_spec(b5r.shape), _full_spec(Wfc.shape),
                  _full_spec(bfcr.shape)],
        out_specs=_row_spec(Wfc.shape[1]),
        out_shape=jax.ShapeDtypeStruct((nrows, Wfc.shape[1]), jnp.float32),
    )(agg5, dinv, b5r, Wfc, bfcr)

    return out[:N]
